# baseline (speedup 1.0000x reference)
import jax
import jax.numpy as jnp
from jax.experimental import pallas as pl

EMB = 256
MAXN = 2048
BSIZE = 8
LENGTHS = (512, 768, 1024, 1024, 1024, 1280, 1280, 1280)
TILE = 256
NBLK = tuple(l // TILE for l in LENGTHS)
SEQ_OF_BLK = tuple(s for s in range(BSIZE) for _ in range(NBLK[s]))
JLOC_OF_BLK = tuple(j for s in range(BSIZE) for j in range(NBLK[s]))
NVALID = sum(NBLK)
FEATD = 2 * EMB + 2


def _lookup(table, i):
    v = jnp.int32(table[-1])
    for k in range(len(table) - 2, -1, -1):
        v = jnp.where(i == k, jnp.int32(table[k]), v)
    return v


def _body(xraw_ref, xmp_ref, meta_ref, w6t_ref, w7t_ref, b6_ref, b7_ref,
          w5a_ref, w5b_ref, b5_ref, actor_ref, critic_ref):
    i = pl.program_id(0)
    jloc = _lookup(JLOC_OF_BLK, i)
    first = jloc == 0

    @pl.when(first)
    def _fill():
        actor_ref[...] = jnp.full((MAXN, 1), -1e20, jnp.float32)

    g = jnp.maximum(
        jnp.dot(xmp_ref[...], w6t_ref[...],
                preferred_element_type=jnp.float32) + b6_ref[...], 0.0)
    l = jnp.maximum(
        jnp.dot(xraw_ref[...], w7t_ref[...],
                preferred_element_type=jnp.float32) + b7_ref[...], 0.0)
    p = (jnp.dot(g, w5a_ref[...], preferred_element_type=jnp.float32)
         + jnp.dot(l, w5b_ref[...], preferred_element_type=jnp.float32)
         + b5_ref[...])
    actor_ref[pl.ds(jloc * TILE, TILE), :] = p[:, 0:1]

    q = jnp.where(meta_ref[:, 1] != 0.0, p[:, 1], -1e20)
    m = jnp.max(q)
    prev = jnp.where(first, -jnp.inf, critic_ref[...])
    critic_ref[...] = jnp.maximum(prev, m)


def kernel(features, W5pi, b5pi, W6pi, b6pi, W7pi, b7pi, W5v, b5v):
    w6t = W6pi.T
    w7t = W7pi.T
    b6r = b6pi.reshape(1, EMB)
    b7r = b7pi.reshape(1, EMB)
    w5t = jnp.concatenate([W5pi, W5v], axis=0).T
    w5a = w5t[:EMB]
    w5b = w5t[EMB:]
    b5 = jnp.stack([b5pi, b5v], axis=1)

    actor_flat, crit3 = pl.pallas_call(
        _body,
        grid=(NVALID,),
        in_specs=[
            pl.BlockSpec((TILE, EMB), lambda i: (i, 0)),
            pl.BlockSpec((TILE, EMB), lambda i: (i, 1)),
            pl.BlockSpec((TILE, 128), lambda i: (i, 4)),
            pl.BlockSpec((EMB, EMB), lambda i: (0, 0)),
            pl.BlockSpec((EMB, EMB), lambda i: (0, 0)),
            pl.BlockSpec((1, EMB), lambda i: (0, 0)),
            pl.BlockSpec((1, EMB), lambda i: (0, 0)),
            pl.BlockSpec((EMB, 2), lambda i: (0, 0)),
            pl.BlockSpec((EMB, 2), lambda i: (0, 0)),
            pl.BlockSpec((1, 2), lambda i: (0, 0)),
        ],
        out_specs=[
            pl.BlockSpec((MAXN, 1), lambda i: (_lookup(SEQ_OF_BLK, i), 0)),
            pl.BlockSpec((1, 1, 1), lambda i: (_lookup(SEQ_OF_BLK, i), 0, 0)),
        ],
        out_shape=[
            jax.ShapeDtypeStruct((BSIZE * MAXN, 1), jnp.float32),
            jax.ShapeDtypeStruct((BSIZE, 1, 1), jnp.float32),
        ],
    )(features, features, features, w6t, w7t, b6r, b7r, w5a, w5b, b5)

    return actor_flat.reshape(BSIZE, MAXN, 1), crit3.reshape(BSIZE, 1)


# speedup vs baseline: 1.0906x; 1.0906x over previous
import jax
import jax.numpy as jnp
from jax.experimental import pallas as pl

EMB = 256
MAXN = 2048
BSIZE = 8
LENGTHS = (512, 768, 1024, 1024, 1024, 1280, 1280, 1280)
TILE = 256
NBLK = tuple(l // TILE for l in LENGTHS)
SEQ_OF_BLK = tuple(s for s in range(BSIZE) for _ in range(NBLK[s]))
JLOC_OF_BLK = tuple(j for s in range(BSIZE) for j in range(NBLK[s]))
NVALID = sum(NBLK)
ROWS_PER_SEQ = MAXN // TILE


def _lookup(table, i):
    v = jnp.int32(table[-1])
    for k in range(len(table) - 2, -1, -1):
        v = jnp.where(i == k, jnp.int32(table[k]), v)
    return v


def _body(xraw_ref, xmp_ref, meta_ref, w6t_ref, w7t_ref, b6_ref, b7_ref,
          w5a_ref, w5b_ref, b5_ref, actor_ref, critic_ref):
    i = pl.program_id(0)
    jloc = _lookup(JLOC_OF_BLK, i)
    first = jloc == 0

    @pl.when(first)
    def _fill():
        actor_ref[...] = jnp.full((ROWS_PER_SEQ, TILE), -1e20, jnp.float32)

    g = jnp.maximum(
        jnp.dot(xmp_ref[...], w6t_ref[...],
                preferred_element_type=jnp.float32) + b6_ref[...], 0.0)
    l = jnp.maximum(
        jnp.dot(xraw_ref[...], w7t_ref[...],
                preferred_element_type=jnp.float32) + b7_ref[...], 0.0)
    p = (jnp.dot(g, w5a_ref[...], preferred_element_type=jnp.float32)
         + jnp.dot(l, w5b_ref[...], preferred_element_type=jnp.float32)
         + b5_ref[...])
    actor_ref[pl.ds(jloc, 1), :] = jnp.transpose(p[:, 0:1])

    q = jnp.where(meta_ref[:, 1] != 0.0, p[:, 1], -1e20)
    m = jnp.max(q)
    prev = jnp.where(first, -jnp.inf, critic_ref[...])
    critic_ref[...] = jnp.maximum(prev, m)


def kernel(features, W5pi, b5pi, W6pi, b6pi, W7pi, b7pi, W5v, b5v):
    w6t = W6pi.T
    w7t = W7pi.T
    b6r = b6pi.reshape(1, EMB)
    b7r = b7pi.reshape(1, EMB)
    w5t = jnp.concatenate([W5pi, W5v], axis=0).T
    w5a = w5t[:EMB]
    w5b = w5t[EMB:]
    b5 = jnp.stack([b5pi, b5v], axis=1)

    actor64, crit3 = pl.pallas_call(
        _body,
        grid=(NVALID,),
        in_specs=[
            pl.BlockSpec((TILE, EMB), lambda i: (i, 0)),
            pl.BlockSpec((TILE, EMB), lambda i: (i, 1)),
            pl.BlockSpec((TILE, 128), lambda i: (i, 4)),
            pl.BlockSpec((EMB, EMB), lambda i: (0, 0)),
            pl.BlockSpec((EMB, EMB), lambda i: (0, 0)),
            pl.BlockSpec((1, EMB), lambda i: (0, 0)),
            pl.BlockSpec((1, EMB), lambda i: (0, 0)),
            pl.BlockSpec((EMB, 2), lambda i: (0, 0)),
            pl.BlockSpec((EMB, 2), lambda i: (0, 0)),
            pl.BlockSpec((1, 2), lambda i: (0, 0)),
        ],
        out_specs=[
            pl.BlockSpec((ROWS_PER_SEQ, TILE),
                         lambda i: (_lookup(SEQ_OF_BLK, i), 0)),
            pl.BlockSpec((1, 1, 1), lambda i: (_lookup(SEQ_OF_BLK, i), 0, 0)),
        ],
        out_shape=[
            jax.ShapeDtypeStruct((BSIZE * ROWS_PER_SEQ, TILE), jnp.float32),
            jax.ShapeDtypeStruct((BSIZE, 1, 1), jnp.float32),
        ],
    )(features, features, features, w6t, w7t, b6r, b7r, w5a, w5b, b5)

    return actor64.reshape(BSIZE, MAXN, 1), crit3.reshape(BSIZE, 1)


# speedup vs baseline: 1.1562x; 1.0602x over previous
import jax
import jax.numpy as jnp
from jax.experimental import pallas as pl
from jax.experimental.pallas import tpu as pltpu

EMB = 256
MAXN = 2048
BSIZE = 8
LENGTHS = (512, 768, 1024, 1024, 1024, 1280, 1280, 1280)
TILE = 256
NBLK = tuple(l // TILE for l in LENGTHS)
SEQ_OF_BLK = tuple(s for s in range(BSIZE) for _ in range(NBLK[s]))
JLOC_OF_BLK = tuple(j for s in range(BSIZE) for j in range(NBLK[s]))
NVALID = sum(NBLK)
ROWS_PER_SEQ = MAXN // TILE


def _lookup(table, i):
    v = jnp.int32(table[-1])
    for k in range(len(table) - 2, -1, -1):
        v = jnp.where(i == k, jnp.int32(table[k]), v)
    return v


def _body(xraw_ref, xmp_ref, meta_ref, w6_ref, w7_ref, w5pi_ref, w5v_ref,
          b6_ref, b7_ref, b5pi_ref, b5v_ref, actor_ref, critic_ref,
          w6t_s, w7t_s, w5a_s, w5b_s):
    i = pl.program_id(0)
    jloc = _lookup(JLOC_OF_BLK, i)
    first = jloc == 0

    @pl.when(i == 0)
    def _prep():
        w6t_s[...] = jnp.transpose(w6_ref[...])
        w7t_s[...] = jnp.transpose(w7_ref[...])
        w5a_s[:, 0:1] = jnp.transpose(w5pi_ref[:, :EMB])
        w5a_s[:, 1:2] = jnp.transpose(w5v_ref[:, :EMB])
        w5b_s[:, 0:1] = jnp.transpose(w5pi_ref[:, EMB:])
        w5b_s[:, 1:2] = jnp.transpose(w5v_ref[:, EMB:])

    @pl.when(first)
    def _fill():
        actor_ref[...] = jnp.full((ROWS_PER_SEQ, TILE), -1e20, jnp.float32)

    g = jnp.maximum(
        jnp.dot(xmp_ref[...], w6t_s[...],
                preferred_element_type=jnp.float32) + b6_ref[...], 0.0)
    l = jnp.maximum(
        jnp.dot(xraw_ref[...], w7t_s[...],
                preferred_element_type=jnp.float32) + b7_ref[...], 0.0)
    p = (jnp.dot(g, w5a_s[...], preferred_element_type=jnp.float32)
         + jnp.dot(l, w5b_s[...], preferred_element_type=jnp.float32))
    actor_ref[pl.ds(jloc, 1), :] = jnp.transpose(p[:, 0:1]) + b5pi_ref[0, 0]

    q = jnp.where(meta_ref[:, 1] != 0.0, p[:, 1] + b5v_ref[0, 0], -1e20)
    m = jnp.max(q)
    prev = jnp.where(first, -jnp.inf, critic_ref[...])
    critic_ref[...] = jnp.maximum(prev, m)


def kernel(features, W5pi, b5pi, W6pi, b6pi, W7pi, b7pi, W5v, b5v):
    actor64, crit3 = pl.pallas_call(
        _body,
        grid=(NVALID,),
        in_specs=[
            pl.BlockSpec((TILE, EMB), lambda i: (i, 0)),
            pl.BlockSpec((TILE, EMB), lambda i: (i, 1)),
            pl.BlockSpec((TILE, 128), lambda i: (i, 4)),
            pl.BlockSpec((EMB, EMB), lambda i: (0, 0)),
            pl.BlockSpec((EMB, EMB), lambda i: (0, 0)),
            pl.BlockSpec((1, 2 * EMB), lambda i: (0, 0)),
            pl.BlockSpec((1, 2 * EMB), lambda i: (0, 0)),
            pl.BlockSpec((1, EMB), lambda i: (0, 0)),
            pl.BlockSpec((1, EMB), lambda i: (0, 0)),
            pl.BlockSpec((1, 1), lambda i: (0, 0)),
            pl.BlockSpec((1, 1), lambda i: (0, 0)),
        ],
        out_specs=[
            pl.BlockSpec((ROWS_PER_SEQ, TILE),
                         lambda i: (_lookup(SEQ_OF_BLK, i), 0)),
            pl.BlockSpec((1, 1, 1), lambda i: (_lookup(SEQ_OF_BLK, i), 0, 0)),
        ],
        out_shape=[
            jax.ShapeDtypeStruct((BSIZE * ROWS_PER_SEQ, TILE), jnp.float32),
            jax.ShapeDtypeStruct((BSIZE, 1, 1), jnp.float32),
        ],
        scratch_shapes=[
            pltpu.VMEM((EMB, EMB), jnp.float32),
            pltpu.VMEM((EMB, EMB), jnp.float32),
            pltpu.VMEM((EMB, 2), jnp.float32),
            pltpu.VMEM((EMB, 2), jnp.float32),
        ],
    )(features, features, features, W6pi, W7pi,
      W5pi, W5v, b6pi.reshape(1, EMB), b7pi.reshape(1, EMB),
      b5pi.reshape(1, 1), b5v.reshape(1, 1))

    return actor64.reshape(BSIZE, MAXN, 1), crit3.reshape(BSIZE, 1)
